# baseline (speedup 1.0000x reference)
import functools

import jax
import jax.numpy as jnp
from jax import lax
from jax.experimental import pallas as pl
from jax.experimental.pallas import tpu as pltpu
from jax.experimental.pallas import tpu_sc as plsc

B = 128
L = 200
V = 100000

NC = 2
TR = 8
NTW = B // TR


def _gather_body(save_id_hbm, rp_hbm, count_hbm, bidx_hbm,
                 colinfo_hbm, tiles_hbm, cnt_out_hbm,
                 sid_v, bidx_v, count_v, colout_v, tile8_v, cntout_v,
                 in_sems, out_sems):
    wid = lax.axis_index("s") * NC + lax.axis_index("c")

    @pl.when(wid < NTW)
    def _work():
        r0 = pl.multiple_of(wid * TR, TR)
        pltpu.sync_copy(bidx_hbm, bidx_v)
        pltpu.sync_copy(count_hbm, count_v)
        pltpu.sync_copy(save_id_hbm.at[pl.ds(r0, TR)], sid_v)

        lane = lax.broadcasted_iota(jnp.int32, (16,), 0)
        valid = lane < TR
        bvs = [bidx_v[pl.ds(k * 16, 16)] for k in range(B // 16)]
        rows = jnp.minimum(lane, TR - 1)
        gcount = plsc.load_gather(count_v, [jnp.minimum(r0 + lane, B - 1)],
                                  mask=valid)
        gcount = jnp.clip(gcount, 0, L - 1)
        col_vec = plsc.load_gather(sid_v, [rows, gcount], mask=valid)

        colfinal = jnp.full((16,), -1, jnp.int32)
        for i in range(TR):
            hit = bvs[0] == (r0 + i)
            for k in range(1, B // 16):
                hit = hit | (bvs[k] == (r0 + i))
            active = jnp.any(hit)
            colfinal = jnp.where((lane == i) & active, col_vec, colfinal)
        colout_v[...] = colfinal
        pltpu.sync_copy(colout_v, colinfo_hbm.at[wid])

        def tile_of(c):
            return pl.multiple_of((c >> 7) << 7, 128)

        cols = [colfinal[j] for j in range(TR)]
        for j in range(TR):
            c = cols[j]

            @pl.when(c >= 0)
            def _():
                pltpu.async_copy(
                    rp_hbm.at[pl.ds(r0, TR), pl.ds(tile_of(c), 128)],
                    tile8_v.at[j], in_sems.at[j]).start()
        for j in range(TR):
            c = cols[j]

            @pl.when(c >= 0)
            def _():
                tc = tile_of(c)
                pltpu.async_copy(
                    rp_hbm.at[pl.ds(r0, TR), pl.ds(tc, 128)],
                    tile8_v.at[j], in_sems.at[j]).wait()
                for j2 in range(TR):
                    c2 = cols[j2]
                    match = (c2 >= 0) & (tile_of(c2) == tc)
                    cl = c2 - tc
                    for g in range(8):
                        v = tile8_v[j, j2, pl.ds(g * 16, 16)]
                        sel = match & ((lane + g * 16) == cl)
                        tile8_v[j, j2, pl.ds(g * 16, 16)] = jnp.where(
                            sel, 1.0, v)
                pltpu.async_copy(tile8_v.at[j], tiles_hbm.at[r0 + j],
                                 out_sems.at[j]).start()
        for j in range(TR):
            c = cols[j]

            @pl.when(c >= 0)
            def _():
                pltpu.async_copy(tile8_v.at[j], tiles_hbm.at[r0 + j],
                                 out_sems.at[j]).wait()

        @pl.when(wid == 0)
        def _cnt():
            for k in range(B // 16):
                cntout_v[pl.ds(k * 16, 16)] = count_v[pl.ds(k * 16, 16)] + 1
            pltpu.sync_copy(cntout_v, cnt_out_hbm)


def _gather_sc(save_id, repeat_penality, penality_reset_count, batch_indices):
    mesh = plsc.VectorSubcoreMesh(core_axis_name="c", subcore_axis_name="s")
    f = pl.kernel(
        _gather_body,
        out_type=[
            jax.ShapeDtypeStruct((NTW, 16), jnp.int32),
            jax.ShapeDtypeStruct((B, TR, 128), jnp.float32),
            jax.ShapeDtypeStruct((B,), jnp.int32),
        ],
        mesh=mesh,
        compiler_params=pltpu.CompilerParams(needs_layout_passes=False),
        scratch_types=[
            pltpu.VMEM((TR, L), jnp.int32),
            pltpu.VMEM((B,), jnp.int32),
            pltpu.VMEM((B,), jnp.int32),
            pltpu.VMEM((16,), jnp.int32),
            pltpu.VMEM((TR, TR, 128), jnp.float32),
            pltpu.VMEM((B,), jnp.int32),
            pltpu.SemaphoreType.DMA((TR,)),
            pltpu.SemaphoreType.DMA((TR,)),
        ],
    )
    return f(save_id, repeat_penality, penality_reset_count, batch_indices)


def _commit_body(rp_ref, colinfo_ref, tiles_ref, out_ref, sems):
    cols = [colinfo_ref[r // TR, r % TR] for r in range(B)]

    def out_cp(r, c):
        rt = (r // TR) * TR
        tc = pl.multiple_of((c >> 7) << 7, 128)
        return pltpu.make_async_copy(
            tiles_ref.at[r], out_ref.at[pl.ds(rt, TR), pl.ds(tc, 128)],
            sems.at[r])

    for r in range(B):
        c = cols[r]

        @pl.when(c >= 0)
        def _():
            out_cp(r, c).start(priority=r % 2)

    for r in range(B):
        c = cols[r]

        @pl.when(c >= 0)
        def _():
            out_cp(r, c).wait()


def _commit_tc(rp_full, colinfo, tiles):
    return pl.pallas_call(
        _commit_body,
        in_specs=[
            pl.BlockSpec(memory_space=pl.ANY),
            pl.BlockSpec(memory_space=pltpu.SMEM),
            pl.BlockSpec(memory_space=pltpu.VMEM),
        ],
        out_specs=pl.BlockSpec(memory_space=pl.ANY),
        out_shape=jax.ShapeDtypeStruct((B, V), jnp.float32),
        input_output_aliases={0: 0},
        scratch_shapes=[
            pltpu.SemaphoreType.DMA((B,)),
        ],
    )(rp_full, colinfo, tiles)


def kernel(save_id, repeat_penality, penality_reset_count, batch_indices):
    colinfo, tiles, cnt_out = _gather_sc(
        save_id, repeat_penality, penality_reset_count, batch_indices)
    zero = lax.optimization_barrier(jnp.zeros((), jnp.float32))
    rp_full = repeat_penality + zero
    rp_out = _commit_tc(rp_full, colinfo, tiles)
    return (save_id, rp_out, cnt_out)


# speedup vs baseline: 1.2280x; 1.2280x over previous
import functools

import jax
import jax.numpy as jnp
from jax import lax
from jax.experimental import pallas as pl
from jax.experimental.pallas import tpu as pltpu
from jax.experimental.pallas import tpu_sc as plsc

B = 128
L = 200
V = 100000

NC = 2
TR = 8
NTW = B // TR


def _gather_body(save_id_hbm, rp_hbm, count_hbm, bidx_hbm,
                 colinfo_hbm, tiles_hbm, cnt_out_hbm,
                 sid_v, bidx_v, count_v, colout_v, tile8_v, cntout_v,
                 in_sems, out_sems):
    wid = lax.axis_index("s") * NC + lax.axis_index("c")

    @pl.when(wid < NTW)
    def _work():
        r0 = pl.multiple_of(wid * TR, TR)
        pltpu.sync_copy(bidx_hbm, bidx_v)
        pltpu.sync_copy(count_hbm, count_v)
        pltpu.sync_copy(save_id_hbm.at[pl.ds(r0, TR)], sid_v)

        lane = lax.broadcasted_iota(jnp.int32, (16,), 0)
        valid = lane < TR
        bvs = [bidx_v[pl.ds(k * 16, 16)] for k in range(B // 16)]
        rows = jnp.minimum(lane, TR - 1)
        gcount = plsc.load_gather(count_v, [jnp.minimum(r0 + lane, B - 1)],
                                  mask=valid)
        gcount = jnp.clip(gcount, 0, L - 1)
        col_vec = plsc.load_gather(sid_v, [rows, gcount], mask=valid)

        colfinal = jnp.full((16,), -1, jnp.int32)
        for i in range(TR):
            hit = bvs[0] == (r0 + i)
            for k in range(1, B // 16):
                hit = hit | (bvs[k] == (r0 + i))
            active = jnp.any(hit)
            colfinal = jnp.where((lane == i) & active, col_vec, colfinal)
        colout_v[...] = colfinal
        pltpu.sync_copy(colout_v, colinfo_hbm.at[wid])

        def tile_of(c):
            return pl.multiple_of((c >> 7) << 7, 128)

        cols = [colfinal[j] for j in range(TR)]
        for j in range(TR):
            c = cols[j]

            @pl.when(c >= 0)
            def _():
                pltpu.async_copy(
                    rp_hbm.at[pl.ds(r0, TR), pl.ds(tile_of(c), 128)],
                    tile8_v.at[j], in_sems.at[j]).start()
        for j in range(TR):
            c = cols[j]

            @pl.when(c >= 0)
            def _():
                tc = tile_of(c)
                pltpu.async_copy(
                    rp_hbm.at[pl.ds(r0, TR), pl.ds(tc, 128)],
                    tile8_v.at[j], in_sems.at[j]).wait()
                for j2 in range(TR):
                    c2 = cols[j2]
                    match = (c2 >= 0) & (tile_of(c2) == tc)
                    cl = c2 - tc
                    for g in range(8):
                        v = tile8_v[j, j2, pl.ds(g * 16, 16)]
                        sel = match & ((lane + g * 16) == cl)
                        tile8_v[j, j2, pl.ds(g * 16, 16)] = jnp.where(
                            sel, 1.0, v)
                pltpu.async_copy(tile8_v.at[j], tiles_hbm.at[r0 + j],
                                 out_sems.at[j]).start()
        for j in range(TR):
            c = cols[j]

            @pl.when(c >= 0)
            def _():
                pltpu.async_copy(tile8_v.at[j], tiles_hbm.at[r0 + j],
                                 out_sems.at[j]).wait()

        @pl.when(wid == 0)
        def _cnt():
            for k in range(B // 16):
                cntout_v[pl.ds(k * 16, 16)] = count_v[pl.ds(k * 16, 16)] + 1
            pltpu.sync_copy(cntout_v, cnt_out_hbm)


def _gather_sc(save_id, repeat_penality, penality_reset_count, batch_indices):
    mesh = plsc.VectorSubcoreMesh(core_axis_name="c", subcore_axis_name="s")
    f = pl.kernel(
        _gather_body,
        out_type=[
            jax.ShapeDtypeStruct((NTW, 16), jnp.int32),
            jax.ShapeDtypeStruct((B, TR, 128), jnp.float32),
            jax.ShapeDtypeStruct((B,), jnp.int32),
        ],
        mesh=mesh,
        compiler_params=pltpu.CompilerParams(needs_layout_passes=False),
        scratch_types=[
            pltpu.VMEM((TR, L), jnp.int32),
            pltpu.VMEM((B,), jnp.int32),
            pltpu.VMEM((B,), jnp.int32),
            pltpu.VMEM((16,), jnp.int32),
            pltpu.VMEM((TR, TR, 128), jnp.float32),
            pltpu.VMEM((B,), jnp.int32),
            pltpu.SemaphoreType.DMA((TR,)),
            pltpu.SemaphoreType.DMA((TR,)),
        ],
    )
    return f(save_id, repeat_penality, penality_reset_count, batch_indices)


def _commit_body(rp_ref, colinfo_ref, tiles_ref, out_ref, sems):
    cols = [colinfo_ref[r // TR, r % TR] for r in range(B)]

    def out_cp(r, c):
        rt = (r // TR) * TR
        tc = pl.multiple_of((c >> 7) << 7, 128)
        return pltpu.make_async_copy(
            tiles_ref.at[r], out_ref.at[pl.ds(rt, TR), pl.ds(tc, 128)],
            sems.at[r])

    for r in range(B):
        c = cols[r]

        @pl.when(c >= 0)
        def _():
            out_cp(r, c).start(priority=r % 2)

    for r in range(B):
        c = cols[r]

        @pl.when(c >= 0)
        def _():
            out_cp(r, c).wait()


def _commit_tc(rp_full, colinfo, tiles):
    return pl.pallas_call(
        _commit_body,
        in_specs=[
            pl.BlockSpec(memory_space=pl.ANY),
            pl.BlockSpec(memory_space=pltpu.SMEM),
            pl.BlockSpec(memory_space=pltpu.VMEM),
        ],
        out_specs=pl.BlockSpec(memory_space=pl.ANY),
        out_shape=jax.ShapeDtypeStruct((B, V), jnp.float32),
        input_output_aliases={0: 0},
        scratch_shapes=[
            pltpu.SemaphoreType.DMA((B,)),
        ],
    )(rp_full, colinfo, tiles)


def kernel(save_id, repeat_penality, penality_reset_count, batch_indices):
    colinfo, tiles, cnt_out = _gather_sc(
        save_id, repeat_penality, penality_reset_count, batch_indices)
    rp_full = repeat_penality + 0.0
    rp_out = _commit_tc(rp_full, colinfo, tiles)
    return (save_id, rp_out, cnt_out)
